# baseline (speedup 1.0000x reference)
import functools

import jax
import jax.numpy as jnp
from jax import lax
from jax.experimental import pallas as pl
from jax.experimental.pallas import tpu as pltpu
from jax.experimental.pallas import tpu_sc as plsc

NC = 2
NS = 16
L = 16
NW = NC * NS

B = 4096
P = 20
N = 50
S = P + N
SP = 72
SPD = 80
D = 128
NT = 4
NTYPES_LEN = 64
EPS = 1e-15

BPW = B // NW
SLOTS = BPW * SP
DSLOTS = BPW * SPD
KD = D // L
NG = SPD // L
NB = 4
BPS = 2
NSTR = BPW // BPS
RBUF = BPS * SP + (SPD - SP)


def _tree_sum(vs):
    while len(vs) > 1:
        vs = [vs[i] + vs[i + 1] for i in range(0, len(vs) - 1, 2)] + (
            [vs[-1]] if len(vs) % 2 else [])
    return vs[0]


def _sc_body(samples_hbm, snode_hbm, types_hbm, semb_hbm, eemb_hbm,
             dots_hbm,
             samp_v, flat_v, snode_v, types_v, t_v, srows_v,
             rows0, rows1, rows2, rows3, dots_v, sem0, sem1, sem2, sem3):
    wid = lax.axis_index("s") * NC + lax.axis_index("c")
    base_b = wid * BPW

    pltpu.sync_copy(samples_hbm.at[pl.ds(wid * SLOTS, SLOTS)], samp_v)
    pltpu.sync_copy(snode_hbm.at[pl.ds(base_b, BPW)], snode_v)
    pltpu.sync_copy(types_hbm, types_v)

    pltpu.async_copy(semb_hbm.at[snode_v], srows_v, sem0).wait()

    for g in range(BPW // L):
        sn = snode_v[pl.ds(g * L, L)]
        t_v[pl.ds(g * L, L)] = plsc.load_gather(types_v, [sn])

    iota = lax.iota(jnp.int32, L)

    def flat_body(i, c):
        basei = i * L
        lanes = basei + iota
        bloc = lax.div(lanes, SP)
        tt = plsc.load_gather(t_v, [bloc])
        sv = samp_v[pl.ds(basei, L)]
        flat_v[pl.ds(basei, L)] = sv * NT + tt
        return c

    lax.fori_loop(0, SLOTS // L, flat_body, 0)

    def fire(s, buf, sem):
        pltpu.async_copy(eemb_hbm.at[flat_v.at[pl.ds(s * BPS * SP, BPS * SP)]],
                         buf.at[pl.ds(0, BPS * SP)], sem)

    def drain(s, buf, sem):
        pltpu.make_async_copy(
            eemb_hbm.at[flat_v.at[pl.ds(s * BPS * SP, BPS * SP)]],
            buf.at[pl.ds(0, BPS * SP)], sem).wait()

    masks = [(iota & k) != 0 for k in (1, 2, 4, 8)]
    perms = [iota ^ k for k in (1, 2, 4, 8)]

    _dnums = lax.GatherDimensionNumbers(
        offset_dims=(), collapsed_slice_dims=(0,), start_index_map=(0,))

    def _shuf(v, r):
        return lax.gather(v, perms[r][:, None], _dnums, slice_sizes=(1,),
                          mode=lax.GatherScatterMode.PROMISE_IN_BOUNDS)

    def compute(b, buf, rbase):
        svecs = [srows_v[b, pl.ds(k * L, L)] for k in range(KD)]

        def group_body(g, c):
            row0 = g * L
            accs = []
            for jj in range(L):
                accs.append(_tree_sum(
                    [buf[rbase + row0 + jj, pl.ds(k * L, L)] * svecs[k]
                     for k in range(KD)]))
            for r in range(4):
                accs = [jnp.where(masks[r], accs[2 * m + 1], accs[2 * m])
                        + _shuf(jnp.where(masks[r], accs[2 * m],
                                          accs[2 * m + 1]), r)
                        for m in range(len(accs) // 2)]
            dots_v[pl.ds(b * SPD + row0, L)] = accs[0]
            return c

        lax.fori_loop(0, NG, group_body, 0)

    rings = (rows0, rows1, rows2, rows3)
    sems = (sem0, sem1, sem2, sem3)
    for q in range(NB):
        fire(q, rings[q], sems[q])

    def ring_body(i, c):
        s0 = NB * i
        for q in range(NB):
            s = s0 + q
            drain(s, rings[q], sems[q])
            for u in range(BPS):
                compute(s * BPS + u, rings[q], u * SP)

            @pl.when(s + NB < NSTR)
            def _():
                fire(s + NB, rings[q], sems[q])
        return c

    lax.fori_loop(0, NSTR // NB, ring_body, 0)

    pltpu.sync_copy(dots_v, dots_hbm.at[pl.ds(wid * DSLOTS, DSLOTS)])


@functools.cache
def _sc_dots_fn():
  return functools.partial(
    pl.kernel,
    out_type=jax.ShapeDtypeStruct((B * SPD,), jnp.float32),
    mesh=plsc.VectorSubcoreMesh(core_axis_name="c", subcore_axis_name="s",
                                num_cores=NC, num_subcores=NS),
    scratch_types=[
        pltpu.VMEM((SLOTS,), jnp.int32),
        pltpu.VMEM((SLOTS,), jnp.int32),
        pltpu.VMEM((BPW,), jnp.int32),
        pltpu.VMEM((NTYPES_LEN,), jnp.int32),
        pltpu.VMEM((BPW,), jnp.int32),
        pltpu.VMEM((BPW, D), jnp.float32),
        pltpu.VMEM((RBUF, D), jnp.float32),
        pltpu.VMEM((RBUF, D), jnp.float32),
        pltpu.VMEM((RBUF, D), jnp.float32),
        pltpu.VMEM((RBUF, D), jnp.float32),
        pltpu.VMEM((DSLOTS,), jnp.float32),
        pltpu.SemaphoreType.DMA,
        pltpu.SemaphoreType.DMA,
        pltpu.SemaphoreType.DMA,
        pltpu.SemaphoreType.DMA,
    ],
    compiler_params=pltpu.CompilerParams(needs_layout_passes=False),
  )(_sc_body)


def _loss_body(dots_ref, out_ref):
    d = dots_ref[...]
    pos = d[:, :P]
    neg = d[:, P:S]
    pos_prob = jax.nn.sigmoid(pos)
    neg_prob = jax.nn.sigmoid(neg)
    lp = -jnp.mean(jnp.log(pos_prob + EPS), axis=1)
    ln = -jnp.mean(jnp.log(1.0 - neg_prob + EPS), axis=1)
    out_ref[...] = lp + ln


_tc_loss = pl.pallas_call(
    _loss_body,
    out_shape=jax.ShapeDtypeStruct((B,), jnp.float32),
)


def kernel(start_node, pos_samples, neg_samples, start_embeds, end_embeds,
           node_types):
    pad = jnp.zeros((B, SP - S), jnp.int32)
    samples = jnp.concatenate([pos_samples, neg_samples, pad], axis=1)
    samples_flat = samples.reshape(-1)
    snode = start_node.reshape(-1)
    eemb_flat = end_embeds.reshape(-1, D)
    dots = _sc_dots_fn()(samples_flat, snode, node_types, start_embeds,
                         eemb_flat)
    return _tc_loss(dots.reshape(B, SPD))


# speedup vs baseline: 1.0079x; 1.0079x over previous
import functools

import jax
import jax.numpy as jnp
from jax import lax
from jax.experimental import pallas as pl
from jax.experimental.pallas import tpu as pltpu
from jax.experimental.pallas import tpu_sc as plsc

NC = 2
NS = 16
L = 16
NW = NC * NS

B = 4096
P = 20
N = 50
S = P + N
SP = 72
SPD = 80
D = 128
NT = 4
NTYPES_LEN = 64
EPS = 1e-15

BPW = B // NW
SLOTS = BPW * SP
DSLOTS = BPW * SPD
KD = D // L
NG = SPD // L
NB = 4
BPS = 1
NSTR = BPW // BPS
RBUF = BPS * SP + (SPD - SP)


def _tree_sum(vs):
    while len(vs) > 1:
        vs = [vs[i] + vs[i + 1] for i in range(0, len(vs) - 1, 2)] + (
            [vs[-1]] if len(vs) % 2 else [])
    return vs[0]


def _sc_body(samples_hbm, snode_hbm, types_hbm, semb_hbm, eemb_hbm,
             dots_hbm,
             samp_v, flat_v, snode_v, types_v, t_v, srows_v,
             rows0, rows1, rows2, rows3, dots_v, sem0, sem1, sem2, sem3):
    wid = lax.axis_index("s") * NC + lax.axis_index("c")
    base_b = wid * BPW

    pltpu.sync_copy(samples_hbm.at[pl.ds(wid * SLOTS, SLOTS)], samp_v)
    pltpu.sync_copy(snode_hbm.at[pl.ds(base_b, BPW)], snode_v)
    pltpu.sync_copy(types_hbm, types_v)

    pltpu.async_copy(semb_hbm.at[snode_v], srows_v, sem0).wait()

    for g in range(BPW // L):
        sn = snode_v[pl.ds(g * L, L)]
        t_v[pl.ds(g * L, L)] = plsc.load_gather(types_v, [sn])

    iota = lax.iota(jnp.int32, L)

    def flat_body(i, c):
        basei = i * L
        lanes = basei + iota
        bloc = lax.div(lanes, SP)
        tt = plsc.load_gather(t_v, [bloc])
        sv = samp_v[pl.ds(basei, L)]
        flat_v[pl.ds(basei, L)] = sv * NT + tt
        return c

    lax.fori_loop(0, SLOTS // L, flat_body, 0)

    def fire(s, buf, sem):
        pltpu.async_copy(eemb_hbm.at[flat_v.at[pl.ds(s * BPS * SP, BPS * SP)]],
                         buf.at[pl.ds(0, BPS * SP)], sem)

    def drain(s, buf, sem):
        pltpu.make_async_copy(
            eemb_hbm.at[flat_v.at[pl.ds(s * BPS * SP, BPS * SP)]],
            buf.at[pl.ds(0, BPS * SP)], sem).wait()

    masks = [(iota & k) != 0 for k in (1, 2, 4, 8)]
    perms = [iota ^ k for k in (1, 2, 4, 8)]

    _dnums = lax.GatherDimensionNumbers(
        offset_dims=(), collapsed_slice_dims=(0,), start_index_map=(0,))

    def _shuf(v, r):
        return lax.gather(v, perms[r][:, None], _dnums, slice_sizes=(1,),
                          mode=lax.GatherScatterMode.PROMISE_IN_BOUNDS)

    def compute(b, buf, rbase):
        svecs = [srows_v[b, pl.ds(k * L, L)] for k in range(KD)]

        def group_body(g, c):
            row0 = g * L
            accs = []
            for jj in range(L):
                accs.append(_tree_sum(
                    [buf[rbase + row0 + jj, pl.ds(k * L, L)] * svecs[k]
                     for k in range(KD)]))
            for r in range(4):
                accs = [jnp.where(masks[r], accs[2 * m + 1], accs[2 * m])
                        + _shuf(jnp.where(masks[r], accs[2 * m],
                                          accs[2 * m + 1]), r)
                        for m in range(len(accs) // 2)]
            dots_v[pl.ds(b * SPD + row0, L)] = accs[0]
            return c

        lax.fori_loop(0, NG, group_body, 0)

    rings = (rows0, rows1, rows2, rows3)
    sems = (sem0, sem1, sem2, sem3)
    for q in range(NB):
        fire(q, rings[q], sems[q])

    def ring_body(i, c):
        s0 = NB * i
        for q in range(NB):
            s = s0 + q
            drain(s, rings[q], sems[q])
            for u in range(BPS):
                compute(s * BPS + u, rings[q], u * SP)

            @pl.when(s + NB < NSTR)
            def _():
                fire(s + NB, rings[q], sems[q])
        return c

    lax.fori_loop(0, NSTR // NB, ring_body, 0)

    pltpu.sync_copy(dots_v, dots_hbm.at[pl.ds(wid * DSLOTS, DSLOTS)])


@functools.cache
def _sc_dots_fn():
  return functools.partial(
    pl.kernel,
    out_type=jax.ShapeDtypeStruct((B * SPD,), jnp.float32),
    mesh=plsc.VectorSubcoreMesh(core_axis_name="c", subcore_axis_name="s",
                                num_cores=NC, num_subcores=NS),
    scratch_types=[
        pltpu.VMEM((SLOTS,), jnp.int32),
        pltpu.VMEM((SLOTS,), jnp.int32),
        pltpu.VMEM((BPW,), jnp.int32),
        pltpu.VMEM((NTYPES_LEN,), jnp.int32),
        pltpu.VMEM((BPW,), jnp.int32),
        pltpu.VMEM((BPW, D), jnp.float32),
        pltpu.VMEM((RBUF, D), jnp.float32),
        pltpu.VMEM((RBUF, D), jnp.float32),
        pltpu.VMEM((RBUF, D), jnp.float32),
        pltpu.VMEM((RBUF, D), jnp.float32),
        pltpu.VMEM((DSLOTS,), jnp.float32),
        pltpu.SemaphoreType.DMA,
        pltpu.SemaphoreType.DMA,
        pltpu.SemaphoreType.DMA,
        pltpu.SemaphoreType.DMA,
    ],
    compiler_params=pltpu.CompilerParams(needs_layout_passes=False),
  )(_sc_body)


def _loss_body(dots_ref, out_ref):
    d = dots_ref[...]
    pos = d[:, :P]
    neg = d[:, P:S]
    pos_prob = jax.nn.sigmoid(pos)
    neg_prob = jax.nn.sigmoid(neg)
    lp = -jnp.mean(jnp.log(pos_prob + EPS), axis=1)
    ln = -jnp.mean(jnp.log(1.0 - neg_prob + EPS), axis=1)
    out_ref[...] = lp + ln


_tc_loss = pl.pallas_call(
    _loss_body,
    out_shape=jax.ShapeDtypeStruct((B,), jnp.float32),
)


def kernel(start_node, pos_samples, neg_samples, start_embeds, end_embeds,
           node_types):
    pad = jnp.zeros((B, SP - S), jnp.int32)
    samples = jnp.concatenate([pos_samples, neg_samples, pad], axis=1)
    samples_flat = samples.reshape(-1)
    snode = start_node.reshape(-1)
    eemb_flat = end_embeds.reshape(-1, D)
    dots = _sc_dots_fn()(samples_flat, snode, node_types, start_embeds,
                         eemb_flat)
    return _tc_loss(dots.reshape(B, SPD))


# speedup vs baseline: 1.0224x; 1.0143x over previous
import functools

import jax
import jax.numpy as jnp
from jax import lax
from jax.experimental import pallas as pl
from jax.experimental.pallas import tpu as pltpu
from jax.experimental.pallas import tpu_sc as plsc

NC = 2
NS = 16
L = 16
NW = NC * NS

B = 4096
P = 20
N = 50
S = P + N
SP = 72
SPD = 80
D = 128
NT = 4
NTYPES_LEN = 64
EPS = 1e-15

BPW = B // NW
SLOTS = BPW * SP
DSLOTS = BPW * SPD
KD = D // L
NG = SPD // L
NB = 4
BPS = 1
NSTR = BPW // BPS
RBUF = BPS * SP + (SPD - SP)
FLAT_HEAD = 2 * NB


def _tree_sum(vs):
    while len(vs) > 1:
        vs = [vs[i] + vs[i + 1] for i in range(0, len(vs) - 1, 2)] + (
            [vs[-1]] if len(vs) % 2 else [])
    return vs[0]


def _sc_body(samples_hbm, snode_hbm, types_hbm, semb_hbm, eemb_hbm,
             dots_hbm,
             samp_v, flat_v, snode_v, types_v, t_v, srows_v,
             rows0, rows1, rows2, rows3, dots_v, sem0, sem1, sem2, sem3):
    wid = lax.axis_index("s") * NC + lax.axis_index("c")
    base_b = wid * BPW

    pltpu.sync_copy(samples_hbm.at[pl.ds(wid * SLOTS, SLOTS)], samp_v)
    pltpu.sync_copy(snode_hbm.at[pl.ds(base_b, BPW)], snode_v)
    pltpu.sync_copy(types_hbm, types_v)

    pltpu.async_copy(semb_hbm.at[snode_v], srows_v, sem0).wait()

    for g in range(BPW // L):
        sn = snode_v[pl.ds(g * L, L)]
        t_v[pl.ds(g * L, L)] = plsc.load_gather(types_v, [sn])

    iota = lax.iota(jnp.int32, L)

    def flat_body(i, c):
        basei = i * L
        lanes = basei + iota
        bloc = lax.div(lanes, SP)
        tt = plsc.load_gather(t_v, [bloc])
        sv = samp_v[pl.ds(basei, L)]
        flat_v[pl.ds(basei, L)] = sv * NT + tt
        return c

    lax.fori_loop(0, FLAT_HEAD * SP // L, flat_body, 0)

    def fire(s, buf, sem):
        pltpu.async_copy(eemb_hbm.at[flat_v.at[pl.ds(s * BPS * SP, BPS * SP)]],
                         buf.at[pl.ds(0, BPS * SP)], sem)

    def drain(s, buf, sem):
        pltpu.make_async_copy(
            eemb_hbm.at[flat_v.at[pl.ds(s * BPS * SP, BPS * SP)]],
            buf.at[pl.ds(0, BPS * SP)], sem).wait()

    masks = [(iota & k) != 0 for k in (1, 2, 4, 8)]
    perms = [iota ^ k for k in (1, 2, 4, 8)]

    _dnums = lax.GatherDimensionNumbers(
        offset_dims=(), collapsed_slice_dims=(0,), start_index_map=(0,))

    def _shuf(v, r):
        return lax.gather(v, perms[r][:, None], _dnums, slice_sizes=(1,),
                          mode=lax.GatherScatterMode.PROMISE_IN_BOUNDS)

    def compute(b, buf, rbase):
        svecs = [srows_v[b, pl.ds(k * L, L)] for k in range(KD)]

        def group_body(g, c):
            row0 = g * L
            accs = []
            for jj in range(L):
                accs.append(_tree_sum(
                    [buf[rbase + row0 + jj, pl.ds(k * L, L)] * svecs[k]
                     for k in range(KD)]))
            for r in range(4):
                accs = [jnp.where(masks[r], accs[2 * m + 1], accs[2 * m])
                        + _shuf(jnp.where(masks[r], accs[2 * m],
                                          accs[2 * m + 1]), r)
                        for m in range(len(accs) // 2)]
            dots_v[pl.ds(b * SPD + row0, L)] = accs[0]
            return c

        lax.fori_loop(0, NG, group_body, 0)

    rings = (rows0, rows1, rows2, rows3)
    sems = (sem0, sem1, sem2, sem3)
    for q in range(NB):
        fire(q, rings[q], sems[q])

    lax.fori_loop(FLAT_HEAD * SP // L, SLOTS // L, flat_body, 0)

    def ring_body(i, c):
        s0 = NB * i
        for q in range(NB):
            s = s0 + q
            drain(s, rings[q], sems[q])
            for u in range(BPS):
                compute(s * BPS + u, rings[q], u * SP)

            @pl.when(s + NB < NSTR)
            def _():
                fire(s + NB, rings[q], sems[q])
        return c

    lax.fori_loop(0, NSTR // NB, ring_body, 0)

    pltpu.sync_copy(dots_v, dots_hbm.at[pl.ds(wid * DSLOTS, DSLOTS)])


@functools.cache
def _sc_dots_fn():
  return functools.partial(
    pl.kernel,
    out_type=jax.ShapeDtypeStruct((B * SPD,), jnp.float32),
    mesh=plsc.VectorSubcoreMesh(core_axis_name="c", subcore_axis_name="s",
                                num_cores=NC, num_subcores=NS),
    scratch_types=[
        pltpu.VMEM((SLOTS,), jnp.int32),
        pltpu.VMEM((SLOTS,), jnp.int32),
        pltpu.VMEM((BPW,), jnp.int32),
        pltpu.VMEM((NTYPES_LEN,), jnp.int32),
        pltpu.VMEM((BPW,), jnp.int32),
        pltpu.VMEM((BPW, D), jnp.float32),
        pltpu.VMEM((RBUF, D), jnp.float32),
        pltpu.VMEM((RBUF, D), jnp.float32),
        pltpu.VMEM((RBUF, D), jnp.float32),
        pltpu.VMEM((RBUF, D), jnp.float32),
        pltpu.VMEM((DSLOTS,), jnp.float32),
        pltpu.SemaphoreType.DMA,
        pltpu.SemaphoreType.DMA,
        pltpu.SemaphoreType.DMA,
        pltpu.SemaphoreType.DMA,
    ],
    compiler_params=pltpu.CompilerParams(needs_layout_passes=False),
  )(_sc_body)


def _loss_body(dots_ref, out_ref):
    d = dots_ref[...]
    pos = d[:, :P]
    neg = d[:, P:S]
    pos_prob = jax.nn.sigmoid(pos)
    neg_prob = jax.nn.sigmoid(neg)
    lp = -jnp.mean(jnp.log(pos_prob + EPS), axis=1)
    ln = -jnp.mean(jnp.log(1.0 - neg_prob + EPS), axis=1)
    out_ref[...] = lp + ln


_tc_loss = pl.pallas_call(
    _loss_body,
    out_shape=jax.ShapeDtypeStruct((B,), jnp.float32),
)


def kernel(start_node, pos_samples, neg_samples, start_embeds, end_embeds,
           node_types):
    pad = jnp.zeros((B, SP - S), jnp.int32)
    samples = jnp.concatenate([pos_samples, neg_samples, pad], axis=1)
    samples_flat = samples.reshape(-1)
    snode = start_node.reshape(-1)
    eemb_flat = end_embeds.reshape(-1, D)
    dots = _sc_dots_fn()(samples_flat, snode, node_types, start_embeds,
                         eemb_flat)
    return _tc_loss(dots.reshape(B, SPD))
